# baseline (speedup 1.0000x reference)
import jax
import jax.numpy as jnp
from jax.experimental import pallas as pl
from jax.experimental.pallas import tpu as pltpu

_EPS = 1e-6


def _ln_body(x_ref, m_ref, w_ref, b_ref, o_ref):
    xb = x_ref[0]
    mean = jnp.mean(xb, axis=0, keepdims=True)
    xc = xb - mean
    var = jnp.mean(xc * xc, axis=0, keepdims=True)
    mask = m_ref[0]
    s = jax.lax.rsqrt(var + _EPS) * mask
    o_ref[0] = xc * s * w_ref[...] + b_ref[...] * mask


def kernel(x, active, ln_weight, ln_bias):
    B, C, H, W = x.shape
    P = H * W
    sh = H // active.shape[2]
    sw = W // active.shape[3]
    a = active[:, 0].astype(jnp.float32)
    mask = jnp.repeat(jnp.repeat(a, sh, axis=1), sw, axis=2)
    mask = (mask != 0.0).astype(jnp.float32).reshape(B, 1, P)

    xr = x.reshape(B, C, P)
    w2 = ln_weight.reshape(C, 1)
    b2 = ln_bias.reshape(C, 1)

    out = pl.pallas_call(
        _ln_body,
        grid=(B,),
        in_specs=[
            pl.BlockSpec((1, C, P), lambda i: (i, 0, 0)),
            pl.BlockSpec((1, 1, P), lambda i: (i, 0, 0)),
            pl.BlockSpec((C, 1), lambda i: (0, 0)),
            pl.BlockSpec((C, 1), lambda i: (0, 0)),
        ],
        out_specs=pl.BlockSpec((1, C, P), lambda i: (i, 0, 0)),
        out_shape=jax.ShapeDtypeStruct((B, C, P), jnp.float32),
        compiler_params=pltpu.CompilerParams(
            dimension_semantics=("parallel",),
        ),
    )(xr, mask, w2, b2)
    return out.reshape(B, C, H, W)
